# Initial kernel scaffold; baseline (speedup 1.0000x reference)
import functools

import jax
import jax.numpy as jnp
import numpy as np
from jax import lax
from jax.experimental import pallas as pl
from jax.experimental.pallas import tpu as pltpu
from jax.experimental.pallas import tpu_sc as plsc

_N = 10000
_E = 320000
_IN = 128
_H = 8
_D = 16
_HD = _H * _D
_NC = 2
_NS = 16
_NW = _NC * _NS
_EPW = _E // _NW
_B = 80
_NCHUNK = _EPW // _B
_RPT = _N // _NS
_BLK = 1000
_NBLK = _N // _BLK

_SCALE = 0.25


def _proj_body(h_ref, wq_ref, wkv_ref, bq_ref, bkv_ref, q_ref, kv_ref):
    hb = h_ref[...]
    q_ref[...] = (
        jnp.dot(hb, wq_ref[...], preferred_element_type=jnp.float32) + bq_ref[...]
    )
    kv_ref[...] = (
        jnp.dot(hb, wkv_ref[...], preferred_element_type=jnp.float32) + bkv_ref[...]
    )


_proj = pl.pallas_call(
    _proj_body,
    grid=(_NBLK,),
    in_specs=[
        pl.BlockSpec((_BLK, _IN), lambda i: (i, 0)),
        pl.BlockSpec((_IN, _HD), lambda i: (0, 0)),
        pl.BlockSpec((_IN, 2 * _HD), lambda i: (0, 0)),
        pl.BlockSpec((1, _HD), lambda i: (0, 0)),
        pl.BlockSpec((1, 2 * _HD), lambda i: (0, 0)),
    ],
    out_specs=[
        pl.BlockSpec((_BLK, _HD), lambda i: (i, 0)),
        pl.BlockSpec((_BLK, 2 * _HD), lambda i: (i, 0)),
    ],
    out_shape=[
        jax.ShapeDtypeStruct((_N, _HD), jnp.float32),
        jax.ShapeDtypeStruct((_N, 2 * _HD), jnp.float32),
    ],
)


@functools.partial(
    pl.kernel,
    out_type=(
        jax.ShapeDtypeStruct((_NC, _N, _HD), jnp.float32),
        jax.ShapeDtypeStruct((_NC, _N, _D), jnp.float32),
    ),
    mesh=plsc.VectorSubcoreMesh(core_axis_name="c", subcore_axis_name="s"),
    scratch_types=[
        pltpu.VMEM((_B,), jnp.int32),
        pltpu.VMEM((_B,), jnp.int32),
        pltpu.VMEM((_B, _HD), jnp.float32),
        pltpu.VMEM((_B, 2 * _HD), jnp.float32),
        pltpu.VMEM((_B, _HD), jnp.float32),
        pltpu.VMEM((_B, _D), jnp.float32),
        pltpu.VMEM_SHARED((_N, _HD), jnp.float32),
        pltpu.VMEM_SHARED((_N, _D), jnp.float32),
        pltpu.SemaphoreType.DMA,
        pltpu.SemaphoreType.DMA,
    ],
)
def _sc_edge(
    q_hbm,
    kv_hbm,
    src_hbm,
    dst_hbm,
    wv_out,
    z_out,
    src_v,
    dst_v,
    qg,
    kvg,
    ov,
    oz,
    acc_wv,
    acc_z,
    sem1,
    sem2,
):
    cid = lax.axis_index("c")
    sid = lax.axis_index("s")
    wid = sid * _NC + cid
    zero = jnp.zeros((_D,), jnp.float32)

    def _zrow(r, _):
        for j in range(_H):
            ov[r, pl.ds(_D * j, _D)] = zero
        oz[r, pl.ds(0, _D)] = zero
        return 0

    lax.fori_loop(0, _B, _zrow, 0)
    row0 = sid * _RPT
    for i in range(7):
        pltpu.sync_copy(ov, acc_wv.at[pl.ds(row0 + i * _B, _B)])
        pltpu.sync_copy(oz, acc_z.at[pl.ds(row0 + i * _B, _B)])
    pltpu.sync_copy(ov, acc_wv.at[pl.ds(row0 + _RPT - _B, _B)])
    pltpu.sync_copy(oz, acc_z.at[pl.ds(row0 + _RPT - _B, _B)])
    plsc.subcore_barrier()

    swap = jnp.bitwise_xor(lax.iota(jnp.int32, _D), _H)
    ebase = wid * _EPW

    def _chunk(c, _):
        off = ebase + c * _B
        pltpu.sync_copy(src_hbm.at[pl.ds(off, _B)], src_v)
        pltpu.sync_copy(dst_hbm.at[pl.ds(off, _B)], dst_v)
        cp1 = pltpu.async_copy(kv_hbm.at[src_v], kvg, sem1)
        cp2 = pltpu.async_copy(q_hbm.at[dst_v], qg, sem2)
        cp1.wait()
        cp2.wait()

        def _edge(ei, _):
            a = kvg[ei, pl.ds(0, _D)] * qg[ei, pl.ds(0, _D)]
            for j in range(1, _H):
                a = a + kvg[ei, pl.ds(_D * j, _D)] * qg[ei, pl.ds(_D * j, _D)]
            a = a + jnp.take(
                a, swap, mode=lax.GatherScatterMode.PROMISE_IN_BOUNDS
            )
            s = jnp.exp(jnp.clip(a * _SCALE, -5.0, 5.0))
            for j in range(_H):
                ov[ei, pl.ds(_D * j, _D)] = kvg[ei, pl.ds(_HD + _D * j, _D)] * s
            oz[ei, pl.ds(0, _D)] = s
            return 0

        lax.fori_loop(0, _B, _edge, 0)
        pltpu.sync_copy(ov, acc_wv.at[dst_v], add=True)
        pltpu.sync_copy(oz, acc_z.at[dst_v], add=True)
        return 0

    lax.fori_loop(0, _NCHUNK, _chunk, 0)
    plsc.subcore_barrier()
    pltpu.sync_copy(acc_wv.at[pl.ds(row0, _RPT)], wv_out.at[cid, pl.ds(row0, _RPT)])
    pltpu.sync_copy(acc_z.at[pl.ds(row0, _RPT)], z_out.at[cid, pl.ds(row0, _RPT)])


def _norm_body(wvp_ref, zp_ref, p_ref, t_ref, o_ref):
    wv = wvp_ref[0] + wvp_ref[1]
    z16 = zp_ref[0] + zp_ref[1]
    num = jnp.dot(wv, p_ref[...], preferred_element_type=jnp.float32)
    den = jnp.dot(z16, t_ref[...], preferred_element_type=jnp.float32)
    o_ref[...] = num / (den + 1e-6)


_norm = pl.pallas_call(
    _norm_body,
    grid=(_NBLK,),
    in_specs=[
        pl.BlockSpec((_NC, _BLK, _HD), lambda i: (0, i, 0)),
        pl.BlockSpec((_NC, _BLK, _D), lambda i: (0, i, 0)),
        pl.BlockSpec((_HD, _HD), lambda i: (0, 0)),
        pl.BlockSpec((_D, _HD), lambda i: (0, 0)),
    ],
    out_specs=pl.BlockSpec((_BLK, _HD), lambda i: (i, 0)),
    out_shape=jax.ShapeDtypeStruct((_N, _HD), jnp.float32),
)

_P_np = np.zeros((_HD, _HD), np.float32)
for _d in range(_D):
    for _h in range(_H):
        _P_np[_d * _H + _h, _h * _D + _d] = 1.0
_T_np = np.zeros((_D, _HD), np.float32)
for _j in range(_D):
    for _k in range(_HD):
        if _j % _H == _k // _D:
            _T_np[_j, _k] = 0.5


def _dmajor(w):
    return w.reshape(_IN, _H, _D).transpose(0, 2, 1).reshape(_IN, _HD)


def _dmajor_b(b):
    return b.reshape(_H, _D).T.reshape(1, _HD)


def kernel(h, e, edge_index, Wq, bq, Wk, bk, We, be, Wv, bv):
    wq_t = _dmajor(Wq)
    wkv = jnp.concatenate([_dmajor(Wk), _dmajor(Wv)], axis=1)
    bq_t = _dmajor_b(bq)
    bkv = jnp.concatenate([_dmajor_b(bk), _dmajor_b(bv)], axis=1)
    q_t, kv = _proj(h, wq_t, wkv, bq_t, bkv)
    src = edge_index[0]
    dst = edge_index[1]
    wvp, zp = _sc_edge(q_t, kv, src, dst)
    out = _norm(wvp, zp, jnp.asarray(_P_np), jnp.asarray(_T_np))
    return out.reshape(_N, _H, _D)


# speedup vs baseline: 27.9103x; 27.9103x over previous
import functools

import jax
import jax.numpy as jnp
import numpy as np
from jax import lax
from jax.experimental import pallas as pl
from jax.experimental.pallas import tpu as pltpu
from jax.experimental.pallas import tpu_sc as plsc

_N = 10000
_E = 320000
_IN = 128
_H = 8
_D = 16
_HD = _H * _D
_ROW = _HD
_ZR = _N // _H
_NC = 2
_NS = 16
_NW = _NC * _NS
_EPW = _E // _NW
_B = 40
_NCHUNK = _EPW // _B
_BLK = 1000
_NBLK = _N // _BLK

_SCALE = 0.25


def _proj_body(h_ref, wq_ref, wkv_ref, bq_ref, bkv_ref, q_ref, kv_ref):
    hb = h_ref[...]
    q_ref[...] = (
        jnp.dot(hb, wq_ref[...], preferred_element_type=jnp.float32) + bq_ref[...]
    )
    kv_ref[...] = (
        jnp.dot(hb, wkv_ref[...], preferred_element_type=jnp.float32) + bkv_ref[...]
    )


_proj = pl.pallas_call(
    _proj_body,
    grid=(_NBLK,),
    in_specs=[
        pl.BlockSpec((_BLK, _IN), lambda i: (i, 0)),
        pl.BlockSpec((_IN, _HD), lambda i: (0, 0)),
        pl.BlockSpec((_IN, 2 * _HD), lambda i: (0, 0)),
        pl.BlockSpec((1, _HD), lambda i: (0, 0)),
        pl.BlockSpec((1, 2 * _HD), lambda i: (0, 0)),
    ],
    out_specs=[
        pl.BlockSpec((_BLK, _HD), lambda i: (i, 0)),
        pl.BlockSpec((_BLK, 2 * _HD), lambda i: (i, 0)),
    ],
    out_shape=[
        jax.ShapeDtypeStruct((_N, _HD), jnp.float32),
        jax.ShapeDtypeStruct((_N, 2 * _HD), jnp.float32),
    ],
)


@functools.partial(
    pl.kernel,
    out_type=(
        jax.ShapeDtypeStruct((_NC, _N, _ROW), jnp.float32),
        jax.ShapeDtypeStruct((_NC, _ZR, _ROW), jnp.float32),
    ),
    mesh=plsc.VectorSubcoreMesh(core_axis_name="c", subcore_axis_name="s"),
    scratch_types=[
        pltpu.VMEM((_B,), jnp.int32),
        pltpu.VMEM((_B,), jnp.int32),
        pltpu.VMEM((_B, _HD), jnp.float32),
        pltpu.VMEM((_B, 2 * _HD), jnp.float32),
        pltpu.VMEM((_B, _ROW), jnp.float32),
        pltpu.VMEM((_B, _ROW), jnp.float32),
        pltpu.VMEM((_B,), jnp.int32),
        pltpu.VMEM_SHARED((_N, _ROW), jnp.float32),
        pltpu.VMEM_SHARED((_ZR, _ROW), jnp.float32),
        pltpu.SemaphoreType.DMA,
        pltpu.SemaphoreType.DMA,
    ],
)
def _sc_edge(
    q_hbm,
    kv_hbm,
    src_hbm,
    dst_hbm,
    out_hbm,
    z_hbm,
    src_v,
    dst_v,
    qg,
    kvg,
    oc,
    ocz,
    dst8_v,
    acc,
    acc_z,
    sem1,
    sem2,
):
    cid = lax.axis_index("c")
    sid = lax.axis_index("s")
    wid = sid * _NC + cid
    zero = jnp.zeros((_D,), jnp.float32)

    def _zrow(r, _):
        for j in range(_ROW // _D):
            oc[r, pl.ds(_D * j, _D)] = zero
        return 0

    lax.fori_loop(0, _B, _zrow, 0)
    row0 = sid * 624
    for i in range(15):
        pltpu.sync_copy(oc, acc.at[pl.ds(row0 + i * _B, _B)])
    pltpu.sync_copy(oc, acc.at[pl.ds(row0 + 624 - _B, _B)])

    @pl.when(sid == _NS - 1)
    def _zero_tail():
        pltpu.sync_copy(oc, acc.at[pl.ds(_N - _B, _B)])

    zrow0 = sid * 80
    pltpu.sync_copy(oc, acc_z.at[pl.ds(zrow0, _B)])

    @pl.when(sid < _NS - 1)
    def _zero_z_hi():
        pltpu.sync_copy(oc, acc_z.at[pl.ds(zrow0 + _B, _B)])

    @pl.when(sid == _NS - 1)
    def _zero_z_tail():
        pltpu.sync_copy(oc.at[pl.ds(0, 10)], acc_z.at[pl.ds(_ZR - 10, 10)])

    plsc.subcore_barrier()

    iot = lax.iota(jnp.int32, _D)
    swap = jnp.bitwise_xor(iot, _H)
    ebase = wid * _EPW

    def _chunk(c, _):
        off = ebase + c * _B
        pltpu.sync_copy(src_hbm.at[pl.ds(off, _B)], src_v)
        pltpu.sync_copy(dst_hbm.at[pl.ds(off, _B)], dst_v)
        cp1 = pltpu.async_copy(kv_hbm.at[src_v], kvg, sem1)
        cp2 = pltpu.async_copy(q_hbm.at[dst_v], qg, sem2)
        cp1.wait()
        cp2.wait()
        for o in (0, 16, 24):
            dst8_v[pl.ds(o, _D)] = lax.shift_right_logical(
                dst_v[pl.ds(o, _D)], 3
            )

        def _edge(ei, _):
            a = kvg[ei, pl.ds(0, _D)] * qg[ei, pl.ds(0, _D)]
            for j in range(1, _H):
                a = a + kvg[ei, pl.ds(_D * j, _D)] * qg[ei, pl.ds(_D * j, _D)]
            a = a + a.at[swap].get(mode="promise_in_bounds")
            s = jnp.exp(jnp.clip(a * _SCALE, -5.0, 5.0))
            for j in range(_H):
                oc[ei, pl.ds(_D * j, _D)] = kvg[ei, pl.ds(_HD + _D * j, _D)] * s
            grp = dst_v[pl.ds(jnp.bitwise_and(ei, -16), _D)]
            lane = jnp.full((_D,), jnp.bitwise_and(ei, 15), jnp.int32)
            dvec = grp.at[lane].get(mode="promise_in_bounds")
            slot = jnp.bitwise_and(dvec, 7)
            one = jnp.ones((_D,), jnp.int32)
            for k in range(_H):
                ind = jnp.maximum(one - jnp.abs(slot - k), 0)
                ocz[ei, pl.ds(_D * k, _D)] = s * ind.astype(jnp.float32)
            return 0

        lax.fori_loop(0, _B, _edge, 0)
        pltpu.sync_copy(oc, acc.at[dst_v], add=True)
        pltpu.sync_copy(ocz, acc_z.at[dst8_v], add=True)
        return 0

    lax.fori_loop(0, _NCHUNK, _chunk, 0)
    plsc.subcore_barrier()
    pltpu.sync_copy(acc.at[pl.ds(row0, 624)], out_hbm.at[cid, pl.ds(row0, 624)])
    @pl.when(sid < _NS - 1)
    def _copy_z_main():
        pltpu.sync_copy(
            acc_z.at[pl.ds(zrow0, 80)], z_hbm.at[cid, pl.ds(zrow0, 80)]
        )

    @pl.when(sid == _NS - 1)
    def _copy_tail():
        pltpu.sync_copy(
            acc.at[pl.ds(_N - 16, 16)], out_hbm.at[cid, pl.ds(_N - 16, 16)]
        )
        pltpu.sync_copy(
            acc_z.at[pl.ds(_ZR - 50, 50)], z_hbm.at[cid, pl.ds(_ZR - 50, 50)]
        )


def _zsum_body(zp_ref, o_ref):
    o_ref[...] = zp_ref[0] + zp_ref[1]


_zsum = pl.pallas_call(
    _zsum_body,
    out_shape=jax.ShapeDtypeStruct((_ZR, _HD), jnp.float32),
)


def _norm_body(p_ref, z_ref, perm_ref, t_ref, o_ref):
    x = p_ref[0] + p_ref[1]
    num = jnp.dot(x, perm_ref[...], preferred_element_type=jnp.float32)
    den = jnp.dot(z_ref[...], t_ref[...], preferred_element_type=jnp.float32)
    o_ref[...] = num / (den + 1e-6)


_norm = pl.pallas_call(
    _norm_body,
    grid=(_NBLK,),
    in_specs=[
        pl.BlockSpec((_NC, _BLK, _ROW), lambda i: (0, i, 0)),
        pl.BlockSpec((_BLK, _D), lambda i: (i, 0)),
        pl.BlockSpec((_HD, _HD), lambda i: (0, 0)),
        pl.BlockSpec((_D, _HD), lambda i: (0, 0)),
    ],
    out_specs=pl.BlockSpec((_BLK, _HD), lambda i: (i, 0)),
    out_shape=jax.ShapeDtypeStruct((_N, _HD), jnp.float32),
)

_P_np = np.zeros((_HD, _HD), np.float32)
for _d in range(_D):
    for _h in range(_H):
        _P_np[_d * _H + _h, _h * _D + _d] = 1.0
_T_np = np.zeros((_D, _HD), np.float32)
for _j in range(_D):
    for _k in range(_HD):
        if _j % _H == _k // _D:
            _T_np[_j, _k] = 0.5


def _dmajor(w):
    return w.reshape(_IN, _H, _D).transpose(0, 2, 1).reshape(_IN, _HD)


def _dmajor_b(b):
    return b.reshape(_H, _D).T.reshape(1, _HD)


def kernel(h, e, edge_index, Wq, bq, Wk, bk, We, be, Wv, bv):
    wq_t = _dmajor(Wq)
    wkv = jnp.concatenate([_dmajor(Wk), _dmajor(Wv)], axis=1)
    bq_t = _dmajor_b(bq)
    bkv = jnp.concatenate([_dmajor_b(bk), _dmajor_b(bv)], axis=1)
    q_t, kv = _proj(h, wq_t, wkv, bq_t, bkv)
    src = edge_index[0]
    dst = edge_index[1]
    acc, zp = _sc_edge(q_t, kv, src, dst)
    z2 = _zsum(zp).reshape(_N, _D)
    out = _norm(acc, z2, jnp.asarray(_P_np), jnp.asarray(_T_np))
    return out.reshape(_N, _H, _D)


# speedup vs baseline: 37.0372x; 1.3270x over previous
import functools

import jax
import jax.numpy as jnp
import numpy as np
from jax import lax
from jax.experimental import pallas as pl
from jax.experimental.pallas import tpu as pltpu
from jax.experimental.pallas import tpu_sc as plsc

_N = 10000
_E = 320000
_IN = 128
_H = 8
_D = 16
_HD = _H * _D
_ZR = _N // 16
_NC = 2
_NS = 16
_NW = _NC * _NS
_EPW = _E // _NW
_B = 40
_NCHUNK = _EPW // _B
_BLK = 1000
_NBLK = _N // _BLK

_SCALE = 0.25


def _proj_body(h_ref, wq_ref, wkv_ref, bq_ref, bkv_ref, q_ref, kv_ref):
    hb = h_ref[...]
    q_ref[...] = (
        jnp.dot(hb, wq_ref[...], preferred_element_type=jnp.float32) + bq_ref[...]
    )
    kv_ref[...] = (
        jnp.dot(hb, wkv_ref[...], preferred_element_type=jnp.float32) + bkv_ref[...]
    )


_proj = pl.pallas_call(
    _proj_body,
    grid=(_NBLK,),
    in_specs=[
        pl.BlockSpec((_BLK, _IN), lambda i: (i, 0)),
        pl.BlockSpec((_IN, _HD), lambda i: (0, 0)),
        pl.BlockSpec((_IN, 2 * _HD), lambda i: (0, 0)),
        pl.BlockSpec((1, _HD), lambda i: (0, 0)),
        pl.BlockSpec((1, 2 * _HD), lambda i: (0, 0)),
    ],
    out_specs=[
        pl.BlockSpec((_BLK, _HD), lambda i: (i, 0)),
        pl.BlockSpec((_BLK, 2 * _HD), lambda i: (i, 0)),
    ],
    out_shape=[
        jax.ShapeDtypeStruct((_N, _HD), jnp.float32),
        jax.ShapeDtypeStruct((_N, 2 * _HD), jnp.float32),
    ],
)


@functools.partial(
    pl.kernel,
    out_type=(
        jax.ShapeDtypeStruct((_NC, _N, _HD), jnp.float32),
        jax.ShapeDtypeStruct((_NC, _ZR, _HD), jnp.float32),
    ),
    mesh=plsc.VectorSubcoreMesh(core_axis_name="c", subcore_axis_name="s"),
    scratch_types=[
        pltpu.VMEM((_B,), jnp.int32),
        pltpu.VMEM((_B,), jnp.int32),
        pltpu.VMEM((_B,), jnp.int32),
        pltpu.VMEM((_B,), jnp.int32),
        pltpu.VMEM((_B,), jnp.int32),
        pltpu.VMEM((_B, _HD), jnp.float32),
        pltpu.VMEM((_B, _HD), jnp.float32),
        pltpu.VMEM((_B, 2 * _HD), jnp.float32),
        pltpu.VMEM((_B, 2 * _HD), jnp.float32),
        pltpu.VMEM((_B, _HD), jnp.float32),
        pltpu.VMEM((_B, _HD), jnp.float32),
        pltpu.VMEM_SHARED((_N, _HD), jnp.float32),
        pltpu.VMEM_SHARED((_ZR, _HD), jnp.float32),
        pltpu.SemaphoreType.DMA,
        pltpu.SemaphoreType.DMA,
        pltpu.SemaphoreType.DMA,
        pltpu.SemaphoreType.DMA,
        pltpu.SemaphoreType.DMA,
        pltpu.SemaphoreType.DMA,
        pltpu.SemaphoreType.DMA,
        pltpu.SemaphoreType.DMA,
    ],
)
def _sc_edge(
    q_hbm,
    kv_hbm,
    src_hbm,
    dst_hbm,
    out_hbm,
    z_hbm,
    src_v0,
    src_v1,
    dst_v0,
    dst_v1,
    dst8_v,
    qg0,
    qg1,
    kvg0,
    kvg1,
    oc,
    ocz,
    acc,
    acc_z,
    sem_s0,
    sem_s1,
    sem_d0,
    sem_d1,
    sem_q0,
    sem_q1,
    sem_kv0,
    sem_kv1,
):
    cid = lax.axis_index("c")
    sid = lax.axis_index("s")
    wid = sid * _NC + cid
    zero = jnp.zeros((_D,), jnp.float32)
    srcs = [src_v0, src_v1]
    dsts = [dst_v0, dst_v1]
    qgs = [qg0, qg1]
    kvgs = [kvg0, kvg1]
    sem_s = [sem_s0, sem_s1]
    sem_d = [sem_d0, sem_d1]
    sem_q = [sem_q0, sem_q1]
    sem_kv = [sem_kv0, sem_kv1]

    def _zrow(r, _):
        for j in range(_H):
            oc[r, pl.ds(_D * j, _D)] = zero
        return 0

    lax.fori_loop(0, _B, _zrow, 0)
    row0 = sid * 624
    zrow0 = sid * 40
    for i in range(15):
        pltpu.sync_copy(oc, acc.at[pl.ds(row0 + i * _B, _B)])
    pltpu.sync_copy(oc, acc.at[pl.ds(row0 + 624 - _B, _B)])

    @pl.when(sid < _NS - 1)
    def _zero_z():
        pltpu.sync_copy(oc, acc_z.at[pl.ds(zrow0, _B)])

    @pl.when(sid == _NS - 1)
    def _zero_tails():
        pltpu.sync_copy(oc, acc.at[pl.ds(_N - _B, _B)])
        pltpu.sync_copy(oc.at[pl.ds(0, 25)], acc_z.at[pl.ds(_ZR - 25, 25)])

    plsc.subcore_barrier()

    iot = lax.iota(jnp.int32, _D)
    swap = jnp.bitwise_xor(iot, _H)
    half = lax.shift_right_logical(iot, 3)
    ebase = wid * _EPW

    def _idx_load(c, b):
        off = ebase + c * _B
        cps = pltpu.async_copy(src_hbm.at[pl.ds(off, _B)], srcs[b], sem_s[b])
        cpd = pltpu.async_copy(dst_hbm.at[pl.ds(off, _B)], dsts[b], sem_d[b])
        return cps, cpd

    def _idx_wait(b):
        pltpu.make_async_copy(
            src_hbm.at[pl.ds(0, _B)], srcs[b], sem_s[b]
        ).wait()
        pltpu.make_async_copy(
            dst_hbm.at[pl.ds(0, _B)], dsts[b], sem_d[b]
        ).wait()

    def _gather_issue(b):
        pltpu.async_copy(kv_hbm.at[srcs[b]], kvgs[b], sem_kv[b])
        pltpu.async_copy(q_hbm.at[dsts[b]], qgs[b], sem_q[b])

    def _gather_wait(b):
        pltpu.make_async_copy(kv_hbm.at[srcs[b]], kvgs[b], sem_kv[b]).wait()
        pltpu.make_async_copy(q_hbm.at[dsts[b]], qgs[b], sem_q[b]).wait()

    cps, cpd = _idx_load(0, 0)
    cps.wait()
    cpd.wait()
    _gather_issue(0)
    _idx_load(1, 1)

    def _step(c, b):
        b1 = 1 - b
        kvg = kvgs[b]
        qg = qgs[b]
        dst_v = dsts[b]

        @pl.when(c + 1 < _NCHUNK)
        def _prefetch():
            _idx_wait(b1)
            _gather_issue(b1)

        _gather_wait(b)

        def _edge(ei, _):
            a = kvg[ei, pl.ds(0, _D)] * qg[ei, pl.ds(0, _D)]
            for j in range(1, _H):
                a = a + kvg[ei, pl.ds(_D * j, _D)] * qg[ei, pl.ds(_D * j, _D)]
            a = a + a.at[swap].get(mode="promise_in_bounds")
            s = jnp.exp(jnp.clip(a * _SCALE, -5.0, 5.0))
            for j in range(_H):
                oc[ei, pl.ds(_D * j, _D)] = kvg[ei, pl.ds(_HD + _D * j, _D)] * s
            grp = dst_v[pl.ds(jnp.bitwise_and(ei, -16), _D)]
            lane = jnp.full((_D,), jnp.bitwise_and(ei, 15), jnp.int32)
            dvec = grp.at[lane].get(mode="promise_in_bounds")
            slot = jnp.bitwise_and(dvec, 15)
            one = jnp.ones((_D,), jnp.int32)
            for k in range(_H):
                tgt = half + 2 * k
                ind = jnp.maximum(one - jnp.abs(slot - tgt), 0)
                ocz[ei, pl.ds(_D * k, _D)] = s * ind.astype(jnp.float32)
            return 0

        lax.fori_loop(0, _B, _edge, 0)
        for o in (0, 16, 24):
            dst8_v[pl.ds(o, _D)] = lax.shift_right_logical(
                dst_v[pl.ds(o, _D)], 4
            )
        pltpu.sync_copy(oc, acc.at[dst_v], add=True)
        pltpu.sync_copy(ocz, acc_z.at[dst8_v], add=True)

        @pl.when(c + 2 < _NCHUNK)
        def _next_idx():
            _idx_load(c + 2, b)

        return 0

    def _pair(i, _):
        _step(2 * i, 0)
        _step(2 * i + 1, 1)
        return 0

    lax.fori_loop(0, _NCHUNK // 2, _pair, 0)
    plsc.subcore_barrier()
    pltpu.sync_copy(acc.at[pl.ds(row0, 624)], out_hbm.at[cid, pl.ds(row0, 624)])

    @pl.when(sid < _NS - 1)
    def _copy_z_main():
        pltpu.sync_copy(
            acc_z.at[pl.ds(zrow0, 40)], z_hbm.at[cid, pl.ds(zrow0, 40)]
        )

    @pl.when(sid == _NS - 1)
    def _copy_tails():
        pltpu.sync_copy(
            acc.at[pl.ds(_N - 16, 16)], out_hbm.at[cid, pl.ds(_N - 16, 16)]
        )
        pltpu.sync_copy(
            acc_z.at[pl.ds(_ZR - 25, 25)], z_hbm.at[cid, pl.ds(_ZR - 25, 25)]
        )


def _zsum_body(zp_ref, o_ref):
    o_ref[...] = zp_ref[0] + zp_ref[1]


_zsum = pl.pallas_call(
    _zsum_body,
    out_shape=jax.ShapeDtypeStruct((_ZR, _HD), jnp.float32),
)


def _norm_body(p_ref, z_ref, perm_ref, t_ref, o_ref):
    x = p_ref[0] + p_ref[1]
    num = jnp.dot(x, perm_ref[...], preferred_element_type=jnp.float32)
    den = jnp.dot(z_ref[...], t_ref[...], preferred_element_type=jnp.float32)
    o_ref[...] = num / (den + 1e-6)


_norm = pl.pallas_call(
    _norm_body,
    grid=(_NBLK,),
    in_specs=[
        pl.BlockSpec((_NC, _BLK, _HD), lambda i: (0, i, 0)),
        pl.BlockSpec((_BLK, _H), lambda i: (i, 0)),
        pl.BlockSpec((_HD, _HD), lambda i: (0, 0)),
        pl.BlockSpec((_H, _HD), lambda i: (0, 0)),
    ],
    out_specs=pl.BlockSpec((_BLK, _HD), lambda i: (i, 0)),
    out_shape=jax.ShapeDtypeStruct((_N, _HD), jnp.float32),
)

_P_np = np.zeros((_HD, _HD), np.float32)
for _d in range(_D):
    for _h in range(_H):
        _P_np[_d * _H + _h, _h * _D + _d] = 1.0
_T_np = np.zeros((_H, _HD), np.float32)
for _j in range(_H):
    for _k in range(_HD):
        if _j == _k // _D:
            _T_np[_j, _k] = 1.0


def _dmajor(w):
    return w.reshape(_IN, _H, _D).transpose(0, 2, 1).reshape(_IN, _HD)


def _dmajor_b(b):
    return b.reshape(_H, _D).T.reshape(1, _HD)


def kernel(h, e, edge_index, Wq, bq, Wk, bk, We, be, Wv, bv):
    wq_t = _dmajor(Wq)
    wkv = jnp.concatenate([_dmajor(Wk), _dmajor(Wv)], axis=1)
    bq_t = _dmajor_b(bq)
    bkv = jnp.concatenate([_dmajor_b(bk), _dmajor_b(bv)], axis=1)
    q_t, kv = _proj(h, wq_t, wkv, bq_t, bkv)
    src = edge_index[0]
    dst = edge_index[1]
    acc, zp = _sc_edge(q_t, kv, src, dst)
    z8 = _zsum(zp).reshape(_N, _H)
    out = _norm(acc, z8, jnp.asarray(_P_np), jnp.asarray(_T_np))
    return out.reshape(_N, _H, _D)


# speedup vs baseline: 63.4616x; 1.7135x over previous
import functools

import jax
import jax.numpy as jnp
import numpy as np
from jax import lax
from jax.experimental import pallas as pl
from jax.experimental.pallas import tpu as pltpu
from jax.experimental.pallas import tpu_sc as plsc

_N = 10000
_E = 320000
_IN = 128
_H = 8
_D = 16
_HD = _H * _D
_ZR = _N // 16
_NC = 2
_NS = 16
_NW = _NC * _NS
_EPW = _E // _NW
_B = 40
_NCHUNK = _EPW // _B
_BLK = 1000
_NBLK = _N // _BLK

_SCALE = 0.25


def _proj_body(h_ref, wq_ref, wkv_ref, bq_ref, bkv_ref, q_ref, kv_ref):
    hb = h_ref[...]
    q_ref[...] = (
        jnp.dot(hb, wq_ref[...], preferred_element_type=jnp.float32) + bq_ref[...]
    )
    kv_ref[...] = (
        jnp.dot(hb, wkv_ref[...], preferred_element_type=jnp.float32) + bkv_ref[...]
    )


_proj = pl.pallas_call(
    _proj_body,
    grid=(_NBLK,),
    in_specs=[
        pl.BlockSpec((_BLK, _IN), lambda i: (i, 0)),
        pl.BlockSpec((_IN, _HD), lambda i: (0, 0)),
        pl.BlockSpec((_IN, 2 * _HD), lambda i: (0, 0)),
        pl.BlockSpec((1, _HD), lambda i: (0, 0)),
        pl.BlockSpec((1, 2 * _HD), lambda i: (0, 0)),
    ],
    out_specs=[
        pl.BlockSpec((_BLK, _HD), lambda i: (i, 0)),
        pl.BlockSpec((_BLK, 2 * _HD), lambda i: (i, 0)),
    ],
    out_shape=[
        jax.ShapeDtypeStruct((_N, _HD), jnp.float32),
        jax.ShapeDtypeStruct((_N, 2 * _HD), jnp.float32),
    ],
)


@functools.partial(
    pl.kernel,
    out_type=(
        jax.ShapeDtypeStruct((_NC, _N, _HD), jnp.float32),
        jax.ShapeDtypeStruct((_NC, _ZR, _HD), jnp.float32),
    ),
    mesh=plsc.VectorSubcoreMesh(core_axis_name="c", subcore_axis_name="s"),
    scratch_types=[
        pltpu.VMEM((_B,), jnp.int32),
        pltpu.VMEM((_B,), jnp.int32),
        pltpu.VMEM((_B,), jnp.int32),
        pltpu.VMEM((_B,), jnp.int32),
        pltpu.VMEM((_B,), jnp.int32),
        pltpu.VMEM((_B, _HD), jnp.float32),
        pltpu.VMEM((_B, _HD), jnp.float32),
        pltpu.VMEM((_B, 2 * _HD), jnp.float32),
        pltpu.VMEM((_B, 2 * _HD), jnp.float32),
        pltpu.VMEM((_B, _HD), jnp.float32),
        pltpu.VMEM((_B, _HD), jnp.float32),
        pltpu.VMEM_SHARED((_N, _HD), jnp.float32),
        pltpu.VMEM_SHARED((_ZR, _HD), jnp.float32),
        pltpu.SemaphoreType.DMA,
        pltpu.SemaphoreType.DMA,
        pltpu.SemaphoreType.DMA,
        pltpu.SemaphoreType.DMA,
        pltpu.SemaphoreType.DMA,
        pltpu.SemaphoreType.DMA,
        pltpu.SemaphoreType.DMA,
        pltpu.SemaphoreType.DMA,
    ],
)
def _sc_edge(
    q_hbm,
    kv_hbm,
    src_hbm,
    dst_hbm,
    out_hbm,
    z_hbm,
    src_v0,
    src_v1,
    dst_v0,
    dst_v1,
    dst8_v,
    qg0,
    qg1,
    kvg0,
    kvg1,
    oc,
    ocz,
    acc,
    acc_z,
    sem_s0,
    sem_s1,
    sem_d0,
    sem_d1,
    sem_q0,
    sem_q1,
    sem_kv0,
    sem_kv1,
):
    cid = lax.axis_index("c")
    sid = lax.axis_index("s")
    wid = sid * _NC + cid
    zero = jnp.zeros((_D,), jnp.float32)
    srcs = [src_v0, src_v1]
    dsts = [dst_v0, dst_v1]
    qgs = [qg0, qg1]
    kvgs = [kvg0, kvg1]
    sem_s = [sem_s0, sem_s1]
    sem_d = [sem_d0, sem_d1]
    sem_q = [sem_q0, sem_q1]
    sem_kv = [sem_kv0, sem_kv1]

    def _zrow(r, _):
        for j in range(_H):
            oc[r, pl.ds(_D * j, _D)] = zero
        return 0

    lax.fori_loop(0, _B, _zrow, 0)
    row0 = sid * 624
    zrow0 = sid * 40
    for i in range(15):
        pltpu.sync_copy(oc, acc.at[pl.ds(row0 + i * _B, _B)])
    pltpu.sync_copy(oc, acc.at[pl.ds(row0 + 624 - _B, _B)])

    @pl.when(sid < _NS - 1)
    def _zero_z():
        pltpu.sync_copy(oc, acc_z.at[pl.ds(zrow0, _B)])

    @pl.when(sid == _NS - 1)
    def _zero_tails():
        pltpu.sync_copy(oc, acc.at[pl.ds(_N - _B, _B)])
        pltpu.sync_copy(oc.at[pl.ds(0, 25)], acc_z.at[pl.ds(_ZR - 25, 25)])

    plsc.subcore_barrier()

    iot = lax.iota(jnp.int32, _D)
    swap = jnp.bitwise_xor(iot, _H)
    half = lax.shift_right_logical(iot, 3)
    ebase = wid * _EPW

    def _idx_load(c, b):
        off = ebase + c * _B
        cps = pltpu.async_copy(src_hbm.at[pl.ds(off, _B)], srcs[b], sem_s[b])
        cpd = pltpu.async_copy(dst_hbm.at[pl.ds(off, _B)], dsts[b], sem_d[b])
        return cps, cpd

    def _idx_wait(b):
        pltpu.make_async_copy(
            src_hbm.at[pl.ds(0, _B)], srcs[b], sem_s[b]
        ).wait()
        pltpu.make_async_copy(
            dst_hbm.at[pl.ds(0, _B)], dsts[b], sem_d[b]
        ).wait()

    def _gather_issue(b):
        pltpu.async_copy(kv_hbm.at[srcs[b]], kvgs[b], sem_kv[b])
        pltpu.async_copy(q_hbm.at[dsts[b]], qgs[b], sem_q[b])

    def _gather_wait(b):
        pltpu.make_async_copy(kv_hbm.at[srcs[b]], kvgs[b], sem_kv[b]).wait()
        pltpu.make_async_copy(q_hbm.at[dsts[b]], qgs[b], sem_q[b]).wait()

    cps, cpd = _idx_load(0, 0)
    cps.wait()
    cpd.wait()
    _gather_issue(0)
    _idx_load(1, 1)

    def _step(c, b):
        b1 = 1 - b
        kvg = kvgs[b]
        qg = qgs[b]
        dst_v = dsts[b]

        @pl.when(c + 1 < _NCHUNK)
        def _prefetch():
            _idx_wait(b1)
            _gather_issue(b1)

        _gather_wait(b)

        @plsc.parallel_loop(0, _B, unroll=4)
        def _edge(ei):
            a = kvg[ei, pl.ds(0, _D)] * qg[ei, pl.ds(0, _D)]
            for j in range(1, _H):
                a = a + kvg[ei, pl.ds(_D * j, _D)] * qg[ei, pl.ds(_D * j, _D)]
            a = a + a.at[swap].get(mode="promise_in_bounds")
            s = jnp.exp(jnp.clip(a * _SCALE, -5.0, 5.0))
            for j in range(_H):
                oc[ei, pl.ds(_D * j, _D)] = kvg[ei, pl.ds(_HD + _D * j, _D)] * s
            grp = dst_v[pl.ds(jnp.bitwise_and(ei, -16), _D)]
            lane = jnp.full((_D,), jnp.bitwise_and(ei, 15), jnp.int32)
            dvec = grp.at[lane].get(mode="promise_in_bounds")
            slot = jnp.bitwise_and(dvec, 15)
            one = jnp.ones((_D,), jnp.int32)
            for k in range(_H):
                tgt = half + 2 * k
                ind = jnp.maximum(one - jnp.abs(slot - tgt), 0)
                ocz[ei, pl.ds(_D * k, _D)] = s * ind.astype(jnp.float32)

        for o in (0, 16, 24):
            dst8_v[pl.ds(o, _D)] = lax.shift_right_logical(
                dst_v[pl.ds(o, _D)], 4
            )
        pltpu.sync_copy(oc, acc.at[dst_v], add=True)
        pltpu.sync_copy(ocz, acc_z.at[dst8_v], add=True)

        @pl.when(c + 2 < _NCHUNK)
        def _next_idx():
            _idx_load(c + 2, b)

        return 0

    def _pair(i, _):
        _step(2 * i, 0)
        _step(2 * i + 1, 1)
        return 0

    lax.fori_loop(0, _NCHUNK // 2, _pair, 0)
    plsc.subcore_barrier()
    pltpu.sync_copy(acc.at[pl.ds(row0, 624)], out_hbm.at[cid, pl.ds(row0, 624)])

    @pl.when(sid < _NS - 1)
    def _copy_z_main():
        pltpu.sync_copy(
            acc_z.at[pl.ds(zrow0, 40)], z_hbm.at[cid, pl.ds(zrow0, 40)]
        )

    @pl.when(sid == _NS - 1)
    def _copy_tails():
        pltpu.sync_copy(
            acc.at[pl.ds(_N - 16, 16)], out_hbm.at[cid, pl.ds(_N - 16, 16)]
        )
        pltpu.sync_copy(
            acc_z.at[pl.ds(_ZR - 25, 25)], z_hbm.at[cid, pl.ds(_ZR - 25, 25)]
        )


def _zsum_body(zp_ref, o_ref):
    o_ref[...] = zp_ref[0] + zp_ref[1]


_zsum = pl.pallas_call(
    _zsum_body,
    out_shape=jax.ShapeDtypeStruct((_ZR, _HD), jnp.float32),
)


def _norm_body(p_ref, z_ref, perm_ref, t_ref, o_ref):
    x = p_ref[0] + p_ref[1]
    num = jnp.dot(x, perm_ref[...], preferred_element_type=jnp.float32)
    den = jnp.dot(z_ref[...], t_ref[...], preferred_element_type=jnp.float32)
    o_ref[...] = num / (den + 1e-6)


_norm = pl.pallas_call(
    _norm_body,
    grid=(_NBLK,),
    in_specs=[
        pl.BlockSpec((_NC, _BLK, _HD), lambda i: (0, i, 0)),
        pl.BlockSpec((_BLK, _H), lambda i: (i, 0)),
        pl.BlockSpec((_HD, _HD), lambda i: (0, 0)),
        pl.BlockSpec((_H, _HD), lambda i: (0, 0)),
    ],
    out_specs=pl.BlockSpec((_BLK, _HD), lambda i: (i, 0)),
    out_shape=jax.ShapeDtypeStruct((_N, _HD), jnp.float32),
)

_P_np = np.zeros((_HD, _HD), np.float32)
for _d in range(_D):
    for _h in range(_H):
        _P_np[_d * _H + _h, _h * _D + _d] = 1.0
_T_np = np.zeros((_H, _HD), np.float32)
for _j in range(_H):
    for _k in range(_HD):
        if _j == _k // _D:
            _T_np[_j, _k] = 1.0


def _dmajor(w):
    return w.reshape(_IN, _H, _D).transpose(0, 2, 1).reshape(_IN, _HD)


def _dmajor_b(b):
    return b.reshape(_H, _D).T.reshape(1, _HD)


def kernel(h, e, edge_index, Wq, bq, Wk, bk, We, be, Wv, bv):
    wq_t = _dmajor(Wq)
    wkv = jnp.concatenate([_dmajor(Wk), _dmajor(Wv)], axis=1)
    bq_t = _dmajor_b(bq)
    bkv = jnp.concatenate([_dmajor_b(bk), _dmajor_b(bv)], axis=1)
    q_t, kv = _proj(h, wq_t, wkv, bq_t, bkv)
    src = edge_index[0]
    dst = edge_index[1]
    acc, zp = _sc_edge(q_t, kv, src, dst)
    z8 = _zsum(zp).reshape(_N, _H)
    out = _norm(acc, z8, jnp.asarray(_P_np), jnp.asarray(_T_np))
    return out.reshape(_N, _H, _D)


# speedup vs baseline: 70.6189x; 1.1128x over previous
import functools

import jax
import jax.numpy as jnp
import numpy as np
from jax import lax
from jax.experimental import pallas as pl
from jax.experimental.pallas import tpu as pltpu
from jax.experimental.pallas import tpu_sc as plsc

_N = 10000
_E = 320000
_IN = 128
_H = 8
_D = 16
_HD = _H * _D
_ZR = _N // 16
_NC = 2
_NS = 16
_NW = _NC * _NS
_EPW = _E // _NW
_B = 40
_NCHUNK = _EPW // _B
_BLK = 1000
_NBLK = _N // _BLK

_SCALE = 0.25


def _proj_body(h_ref, wq_ref, wkv_ref, bq_ref, bkv_ref, q_ref, kv_ref):
    hb = h_ref[...]
    q_ref[...] = (
        jnp.dot(hb, wq_ref[...], preferred_element_type=jnp.float32) + bq_ref[...]
    )
    kv_ref[...] = (
        jnp.dot(hb, wkv_ref[...], preferred_element_type=jnp.float32) + bkv_ref[...]
    )


_proj = pl.pallas_call(
    _proj_body,
    grid=(_NBLK,),
    in_specs=[
        pl.BlockSpec((_BLK, _IN), lambda i: (i, 0)),
        pl.BlockSpec((_IN, _HD), lambda i: (0, 0)),
        pl.BlockSpec((_IN, 2 * _HD), lambda i: (0, 0)),
        pl.BlockSpec((1, _HD), lambda i: (0, 0)),
        pl.BlockSpec((1, 2 * _HD), lambda i: (0, 0)),
    ],
    out_specs=[
        pl.BlockSpec((_BLK, _HD), lambda i: (i, 0)),
        pl.BlockSpec((_BLK, 2 * _HD), lambda i: (i, 0)),
    ],
    out_shape=[
        jax.ShapeDtypeStruct((_N, _HD), jnp.float32),
        jax.ShapeDtypeStruct((_N, 2 * _HD), jnp.float32),
    ],
)


@functools.partial(
    pl.kernel,
    out_type=(
        jax.ShapeDtypeStruct((_NC, _N, _HD), jnp.float32),
        jax.ShapeDtypeStruct((_NC, _ZR, _HD), jnp.float32),
    ),
    mesh=plsc.VectorSubcoreMesh(core_axis_name="c", subcore_axis_name="s"),
    scratch_types=[
        pltpu.VMEM((_B,), jnp.int32),
        pltpu.VMEM((_B,), jnp.int32),
        pltpu.VMEM((_B,), jnp.int32),
        pltpu.VMEM((_B,), jnp.int32),
        pltpu.VMEM((_B,), jnp.int32),
        pltpu.VMEM((_B, _HD), jnp.float32),
        pltpu.VMEM((_B, _HD), jnp.float32),
        pltpu.VMEM((_B, 2 * _HD), jnp.float32),
        pltpu.VMEM((_B, 2 * _HD), jnp.float32),
        pltpu.VMEM((_B, _HD), jnp.float32),
        pltpu.VMEM((_B, _HD), jnp.float32),
        pltpu.VMEM_SHARED((_N, _HD), jnp.float32),
        pltpu.VMEM_SHARED((_ZR, _HD), jnp.float32),
        pltpu.SemaphoreType.DMA,
        pltpu.SemaphoreType.DMA,
        pltpu.SemaphoreType.DMA,
        pltpu.SemaphoreType.DMA,
        pltpu.SemaphoreType.DMA,
        pltpu.SemaphoreType.DMA,
        pltpu.SemaphoreType.DMA,
        pltpu.SemaphoreType.DMA,
    ],
)
def _sc_edge(
    q_hbm,
    kv_hbm,
    src_hbm,
    dst_hbm,
    out_hbm,
    z_hbm,
    src_v0,
    src_v1,
    dst_v0,
    dst_v1,
    dst8_v,
    qg0,
    qg1,
    kvg0,
    kvg1,
    oc,
    ocz,
    acc,
    acc_z,
    sem_s0,
    sem_s1,
    sem_d0,
    sem_d1,
    sem_q0,
    sem_q1,
    sem_kv0,
    sem_kv1,
):
    cid = lax.axis_index("c")
    sid = lax.axis_index("s")
    wid = sid * _NC + cid
    zero = jnp.zeros((_D,), jnp.float32)
    srcs = [src_v0, src_v1]
    dsts = [dst_v0, dst_v1]
    qgs = [qg0, qg1]
    kvgs = [kvg0, kvg1]
    sem_s = [sem_s0, sem_s1]
    sem_d = [sem_d0, sem_d1]
    sem_q = [sem_q0, sem_q1]
    sem_kv = [sem_kv0, sem_kv1]

    def _zrow(r, _):
        for j in range(_H):
            oc[r, pl.ds(_D * j, _D)] = zero
        return 0

    lax.fori_loop(0, _B, _zrow, 0)
    row0 = sid * 624
    zrow0 = sid * 40
    for i in range(15):
        pltpu.sync_copy(oc, acc.at[pl.ds(row0 + i * _B, _B)])
    pltpu.sync_copy(oc, acc.at[pl.ds(row0 + 624 - _B, _B)])

    @pl.when(sid < _NS - 1)
    def _zero_z():
        pltpu.sync_copy(oc, acc_z.at[pl.ds(zrow0, _B)])

    @pl.when(sid == _NS - 1)
    def _zero_tails():
        pltpu.sync_copy(oc, acc.at[pl.ds(_N - _B, _B)])
        pltpu.sync_copy(oc.at[pl.ds(0, 25)], acc_z.at[pl.ds(_ZR - 25, 25)])

    plsc.subcore_barrier()

    iot = lax.iota(jnp.int32, _D)
    swap = jnp.bitwise_xor(iot, _H)
    half = lax.shift_right_logical(iot, 3)
    ebase = wid * _EPW

    def _idx_load(c, b):
        off = ebase + c * _B
        cps = pltpu.async_copy(src_hbm.at[pl.ds(off, _B)], srcs[b], sem_s[b])
        cpd = pltpu.async_copy(dst_hbm.at[pl.ds(off, _B)], dsts[b], sem_d[b])
        return cps, cpd

    def _idx_wait(b):
        pltpu.make_async_copy(
            src_hbm.at[pl.ds(0, _B)], srcs[b], sem_s[b]
        ).wait()
        pltpu.make_async_copy(
            dst_hbm.at[pl.ds(0, _B)], dsts[b], sem_d[b]
        ).wait()

    def _gather_issue(b):
        pltpu.async_copy(kv_hbm.at[srcs[b]], kvgs[b], sem_kv[b])
        pltpu.async_copy(q_hbm.at[dsts[b]], qgs[b], sem_q[b])

    def _gather_wait(b):
        pltpu.make_async_copy(kv_hbm.at[srcs[b]], kvgs[b], sem_kv[b]).wait()
        pltpu.make_async_copy(q_hbm.at[dsts[b]], qgs[b], sem_q[b]).wait()

    cps, cpd = _idx_load(0, 0)
    cps.wait()
    cpd.wait()
    _gather_issue(0)
    _idx_load(1, 1)

    def _step(c, b):
        b1 = 1 - b
        kvg = kvgs[b]
        qg = qgs[b]
        dst_v = dsts[b]

        @pl.when(c + 1 < _NCHUNK)
        def _prefetch():
            _idx_wait(b1)
            _gather_issue(b1)

        _gather_wait(b)

        @plsc.parallel_loop(0, _B, unroll=8)
        def _edge(ei):
            a = kvg[ei, pl.ds(0, _D)] * qg[ei, pl.ds(0, _D)]
            for j in range(1, _H):
                a = a + kvg[ei, pl.ds(_D * j, _D)] * qg[ei, pl.ds(_D * j, _D)]
            a = a + a.at[swap].get(mode="promise_in_bounds")
            s = jnp.exp(jnp.clip(a * _SCALE, -5.0, 5.0))
            for j in range(_H):
                oc[ei, pl.ds(_D * j, _D)] = kvg[ei, pl.ds(_HD + _D * j, _D)] * s
            grp = dst_v[pl.ds(jnp.bitwise_and(ei, -16), _D)]
            lane = jnp.full((_D,), jnp.bitwise_and(ei, 15), jnp.int32)
            dvec = grp.at[lane].get(mode="promise_in_bounds")
            slot = jnp.bitwise_and(dvec, 15)
            one = jnp.ones((_D,), jnp.int32)
            for k in range(_H):
                tgt = half + 2 * k
                ind = jnp.maximum(one - jnp.abs(slot - tgt), 0)
                ocz[ei, pl.ds(_D * k, _D)] = s * ind.astype(jnp.float32)

        for o in (0, 16, 24):
            dst8_v[pl.ds(o, _D)] = lax.shift_right_logical(
                dst_v[pl.ds(o, _D)], 4
            )
        pltpu.sync_copy(oc, acc.at[dst_v], add=True)
        pltpu.sync_copy(ocz, acc_z.at[dst8_v], add=True)

        @pl.when(c + 2 < _NCHUNK)
        def _next_idx():
            _idx_load(c + 2, b)

        return 0

    def _pair(i, _):
        _step(2 * i, 0)
        _step(2 * i + 1, 1)
        return 0

    lax.fori_loop(0, _NCHUNK // 2, _pair, 0)
    plsc.subcore_barrier()
    pltpu.sync_copy(acc.at[pl.ds(row0, 624)], out_hbm.at[cid, pl.ds(row0, 624)])

    @pl.when(sid < _NS - 1)
    def _copy_z_main():
        pltpu.sync_copy(
            acc_z.at[pl.ds(zrow0, 40)], z_hbm.at[cid, pl.ds(zrow0, 40)]
        )

    @pl.when(sid == _NS - 1)
    def _copy_tails():
        pltpu.sync_copy(
            acc.at[pl.ds(_N - 16, 16)], out_hbm.at[cid, pl.ds(_N - 16, 16)]
        )
        pltpu.sync_copy(
            acc_z.at[pl.ds(_ZR - 25, 25)], z_hbm.at[cid, pl.ds(_ZR - 25, 25)]
        )


def _zsum_body(zp_ref, o_ref):
    o_ref[...] = zp_ref[0] + zp_ref[1]


_zsum = pl.pallas_call(
    _zsum_body,
    out_shape=jax.ShapeDtypeStruct((_ZR, _HD), jnp.float32),
)


def _norm_body(p_ref, z_ref, perm_ref, t_ref, o_ref):
    x = p_ref[0] + p_ref[1]
    num = jnp.dot(x, perm_ref[...], preferred_element_type=jnp.float32)
    den = jnp.dot(z_ref[...], t_ref[...], preferred_element_type=jnp.float32)
    o_ref[...] = num / (den + 1e-6)


_norm = pl.pallas_call(
    _norm_body,
    grid=(_NBLK,),
    in_specs=[
        pl.BlockSpec((_NC, _BLK, _HD), lambda i: (0, i, 0)),
        pl.BlockSpec((_BLK, _H), lambda i: (i, 0)),
        pl.BlockSpec((_HD, _HD), lambda i: (0, 0)),
        pl.BlockSpec((_H, _HD), lambda i: (0, 0)),
    ],
    out_specs=pl.BlockSpec((_BLK, _HD), lambda i: (i, 0)),
    out_shape=jax.ShapeDtypeStruct((_N, _HD), jnp.float32),
)

_P_np = np.zeros((_HD, _HD), np.float32)
for _d in range(_D):
    for _h in range(_H):
        _P_np[_d * _H + _h, _h * _D + _d] = 1.0
_T_np = np.zeros((_H, _HD), np.float32)
for _j in range(_H):
    for _k in range(_HD):
        if _j == _k // _D:
            _T_np[_j, _k] = 1.0


def _dmajor(w):
    return w.reshape(_IN, _H, _D).transpose(0, 2, 1).reshape(_IN, _HD)


def _dmajor_b(b):
    return b.reshape(_H, _D).T.reshape(1, _HD)


def kernel(h, e, edge_index, Wq, bq, Wk, bk, We, be, Wv, bv):
    wq_t = _dmajor(Wq)
    wkv = jnp.concatenate([_dmajor(Wk), _dmajor(Wv)], axis=1)
    bq_t = _dmajor_b(bq)
    bkv = jnp.concatenate([_dmajor_b(bk), _dmajor_b(bv)], axis=1)
    q_t, kv = _proj(h, wq_t, wkv, bq_t, bkv)
    src = edge_index[0]
    dst = edge_index[1]
    acc, zp = _sc_edge(q_t, kv, src, dst)
    z8 = _zsum(zp).reshape(_N, _H)
    out = _norm(acc, z8, jnp.asarray(_P_np), jnp.asarray(_T_np))
    return out.reshape(_N, _H, _D)
